# baseline (speedup 1.0000x reference)
import functools
import jax
import jax.numpy as jnp
from jax import lax
from jax.experimental import pallas as pl
from jax.experimental.pallas import tpu as pltpu
from jax.experimental.pallas import tpu_sc as plsc

_NB = 128
_NG = _NB // 16
_CH = 16
_DS = 8
_CS = 2


def kernel(x):
    B, C, D = x.shape
    F = C * D
    xt = jnp.transpose(x, (1, 2, 0))
    info = plsc.get_sparse_core_info()
    NC, NS = info.num_cores, info.num_subcores
    NW = NC * NS
    b_per_w = B // NW
    nd = D // _DS
    nk = C // _CS
    mesh = plsc.VectorSubcoreMesh(core_axis_name="c", subcore_axis_name="s")

    @functools.partial(
        pl.kernel,
        mesh=mesh,
        out_type=[
            jax.ShapeDtypeStruct((D, B), jnp.float32),
            jax.ShapeDtypeStruct((B, F), jnp.float32),
        ],
        scratch_types=[
            pltpu.VMEM((_CH, _DS, _NB), jnp.float32),
            pltpu.VMEM((_CH, _DS, _NB), jnp.float32),
            pltpu.VMEM((_CS, D, _NB), jnp.float32),
            pltpu.VMEM((_CS, D, _NB), jnp.float32),
            pltpu.VMEM((_NB, _CS * D), jnp.float32),
            pltpu.VMEM((_NB, _CS * D), jnp.float32),
            pltpu.VMEM((D, _NB), jnp.float32),
            pltpu.VMEM((_NG, 16), jnp.int32),
            pltpu.SemaphoreType.DMA,
            pltpu.SemaphoreType.DMA,
            pltpu.SemaphoreType.DMA,
            pltpu.SemaphoreType.DMA,
            pltpu.SemaphoreType.DMA,
            pltpu.SemaphoreType.DMA,
        ],
        compiler_params=pltpu.CompilerParams(
            use_tc_tiling_on_sc=True, needs_layout_passes=False),
    )
    def run(xt_hbm, logt_hbm, lat_hbm, xa0, xa1, xb0, xb1, nb0, nb1,
            s_buf, gi_buf, sa0, sa1, sb0, sb1, so0, so1):
        wid = lax.axis_index("s") * NC + lax.axis_index("c")
        base = wid * b_per_w
        lanes = lax.iota(jnp.int32, 16)
        zero16 = jnp.zeros((16,), jnp.float32)

        def a_src(b0, j, ch):
            return xt_hbm.at[pl.ds(ch * _CH, _CH),
                             pl.ds(j * _DS, _DS), pl.ds(b0, _NB)]

        def b_src(b0, k):
            return xt_hbm.at[pl.ds(k * _CS, _CS), :, pl.ds(b0, _NB)]

        def lat_dst(b0, k):
            return lat_hbm.at[pl.ds(b0, _NB), pl.ds(k * (_CS * D), _CS * D)]

        def znb(r, carry):
            for j in range(_CS * D // 16):
                nb0[r, pl.ds(j * 16, 16)] = zero16
                nb1[r, pl.ds(j * 16, 16)] = zero16
            return carry

        lax.fori_loop(0, _NB, znb, 0)

        def fill_nb(nb, xb):
            for cl in range(_CS):
                clv = jnp.full((16,), cl, jnp.int32)
                for g in range(_NG):
                    giv = gi_buf[g, :]
                    rows = lanes + g * 16
                    vals = plsc.load_gather(xb, [clv, giv, rows])
                    plsc.store_scatter(nb, [rows, giv + cl * D], vals)

        def chunk(i, carry):
            b0 = base + i * _NB

            pltpu.async_copy(a_src(b0, 0, 0), xa0, sa0)

            def half(buf, j, ch):
                d0 = j * _DS
                for dl in range(_DS):
                    d = d0 + dl
                    for g in range(_NG):
                        sl = pl.ds(g * 16, 16)
                        v0 = buf[0, dl, sl]
                        v1 = buf[1, dl, sl]
                        v2 = buf[2, dl, sl]
                        v3 = buf[3, dl, sl]
                        a0 = v0 * v0
                        a1 = v1 * v1
                        a2 = v2 * v2
                        a3 = v3 * v3
                        for c in range(4, _CH, 4):
                            v0 = buf[c, dl, sl]
                            v1 = buf[c + 1, dl, sl]
                            v2 = buf[c + 2, dl, sl]
                            v3 = buf[c + 3, dl, sl]
                            a0 = a0 + v0 * v0
                            a1 = a1 + v1 * v1
                            a2 = a2 + v2 * v2
                            a3 = a3 + v3 * v3
                        acc = (a0 + a1) + (a2 + a3)
                        if ch == 0:
                            s_buf[d, sl] = acc
                        else:
                            s_buf[d, sl] = s_buf[d, sl] + acc

            def aj(j, carry2):
                pltpu.async_copy(a_src(b0, j, 1), xa1, sa1)
                pltpu.make_async_copy(a_src(b0, j, 0), xa0, sa0).wait()
                half(xa0, j, 0)

                @pl.when(j < nd - 1)
                def _():
                    pltpu.async_copy(a_src(b0, j + 1, 0), xa0, sa0)

                pltpu.make_async_copy(a_src(b0, j, 1), xa1, sa1).wait()
                half(xa1, j, 1)
                return carry2

            lax.fori_loop(0, nd, aj, 0)

            pltpu.async_copy(b_src(b0, 0), xb0, sb0)

            for g in range(_NG):
                sl = pl.ds(g * 16, 16)
                ss = [s_buf[d, sl] for d in range(D)]
                m = ss[0]
                for d in range(1, D):
                    m = jnp.maximum(m, ss[d])
                cand = jnp.full((16,), D, jnp.int32)
                for d in range(D - 1, -1, -1):
                    cand = jnp.where(ss[d] == m, d, cand)
                gi_buf[g, :] = cand

            def nl(d, c3):
                for g in range(_NG):
                    sl = pl.ds(g * 16, 16)
                    acc = s_buf[d, sl]
                    iv = lax.bitcast_convert_type(acc, jnp.int32)
                    y = lax.bitcast_convert_type(
                        jnp.int32(0x5F3759DF) - (iv >> 1), jnp.float32)
                    for _ in range(2):
                        y = y * (1.5 - 0.5 * acc * y * y)
                    s_buf[d, sl] = jnp.where(acc > 0.0, acc * y, 0.0)
                return c3

            lax.fori_loop(0, D, nl, 0)
            pltpu.sync_copy(s_buf, logt_hbm.at[:, pl.ds(b0, _NB)])

            def bj(jj, carry2):
                k0 = 2 * jj
                pltpu.async_copy(b_src(b0, k0 + 1), xb1, sb1)
                pltpu.make_async_copy(b_src(b0, k0), xb0, sb0).wait()

                @pl.when(jj > 0)
                def _():
                    pltpu.make_async_copy(nb0, lat_dst(b0, k0 - 2),
                                          so0).wait()

                fill_nb(nb0, xb0)
                pltpu.async_copy(nb0, lat_dst(b0, k0), so0)

                @pl.when(jj < nk // 2 - 1)
                def _():
                    pltpu.async_copy(b_src(b0, k0 + 2), xb0, sb0)

                pltpu.make_async_copy(b_src(b0, k0 + 1), xb1, sb1).wait()

                @pl.when(jj > 0)
                def _():
                    pltpu.make_async_copy(nb1, lat_dst(b0, k0 - 1),
                                          so1).wait()

                fill_nb(nb1, xb1)
                pltpu.async_copy(nb1, lat_dst(b0, k0 + 1), so1)
                return carry2

            lax.fori_loop(0, nk // 2, bj, 0)

            pltpu.make_async_copy(nb0, lat_dst(b0, nk - 2), so0).wait()
            pltpu.make_async_copy(nb1, lat_dst(b0, nk - 1), so1).wait()
            for g in range(_NG):
                giv = gi_buf[g, :]
                rows = lanes + g * 16
                for cl in range(_CS):
                    plsc.store_scatter(nb0, [rows, giv + cl * D], zero16)
                    plsc.store_scatter(nb1, [rows, giv + cl * D], zero16)
            return carry

        lax.fori_loop(0, b_per_w // _NB, chunk, 0)

    logt, lat = run(xt)
    return (jnp.transpose(logt), lat)


# speedup vs baseline: 1.2351x; 1.2351x over previous
import functools
import jax
import jax.numpy as jnp
from jax import lax
from jax.experimental import pallas as pl
from jax.experimental.pallas import tpu as pltpu
from jax.experimental.pallas import tpu_sc as plsc

_NB = 128
_NG = _NB // 16
_CH = 16
_DS = 8


def kernel(x):
    B, C, D = x.shape
    F = C * D
    xt = jnp.transpose(x, (1, 2, 0))
    info = plsc.get_sparse_core_info()
    NC, NS = info.num_cores, info.num_subcores
    NW = NC * NS
    b_per_w = B // NW
    nd = D // _DS
    nblk = F // _NB
    mesh = plsc.VectorSubcoreMesh(core_axis_name="c", subcore_axis_name="s")

    @functools.partial(
        pl.kernel,
        mesh=mesh,
        out_type=[
            jax.ShapeDtypeStruct((D, B), jnp.float32),
            jax.ShapeDtypeStruct((B, F), jnp.float32),
        ],
        scratch_types=[
            pltpu.VMEM((_CH, _DS, _NB), jnp.float32),
            pltpu.VMEM((_CH, _DS, _NB), jnp.float32),
            pltpu.VMEM((_CH, _DS, _NB), jnp.float32),
            pltpu.VMEM((_CH, _DS, _NB), jnp.float32),
            pltpu.VMEM((_NB, _NB), jnp.float32),
            pltpu.VMEM((_NB, _NB), jnp.float32),
            pltpu.VMEM((D, _NB), jnp.float32),
            pltpu.VMEM((C, _NB), jnp.float32),
            pltpu.VMEM((_NG, 16), jnp.float32),
            pltpu.VMEM((_NG, 16), jnp.int32),
            pltpu.SemaphoreType.DMA,
            pltpu.SemaphoreType.DMA,
            pltpu.SemaphoreType.DMA,
            pltpu.SemaphoreType.DMA,
            pltpu.SemaphoreType.DMA,
            pltpu.SemaphoreType.DMA,
        ],
        compiler_params=pltpu.CompilerParams(
            use_tc_tiling_on_sc=True, needs_layout_passes=False),
    )
    def run(xt_hbm, logt_hbm, lat_hbm, A0, A1, A2, A3, nb0, nb1,
            s_buf, best_buf, m_buf, gi_buf, s0, s1, s2, s3, so0, so1):
        wid = lax.axis_index("s") * NC + lax.axis_index("c")
        base = wid * b_per_w
        lanes = lax.iota(jnp.int32, 16)
        zero16 = jnp.zeros((16,), jnp.float32)

        def a_src(b0, j, ch):
            return xt_hbm.at[pl.ds(ch * _CH, _CH),
                             pl.ds(j * _DS, _DS), pl.ds(b0, _NB)]

        def lat_dst(b0, k):
            return lat_hbm.at[pl.ds(b0, _NB), pl.ds(k * _NB, _NB)]

        def znb(r, carry):
            for j in range(_NB // 16):
                nb0[r, pl.ds(j * 16, 16)] = zero16
                nb1[r, pl.ds(j * 16, 16)] = zero16
            return carry

        lax.fori_loop(0, _NB, znb, 0)

        def half(buf, j, ch):
            d0 = j * _DS

            def dl_body(dl2, c3):
                for du in range(2):
                    dl = dl2 * 2 + du
                    d = d0 + dl
                    for g in range(_NG):
                        sl = pl.ds(g * 16, 16)
                        v0 = buf[0, dl, sl]
                        v1 = buf[1, dl, sl]
                        v2 = buf[2, dl, sl]
                        v3 = buf[3, dl, sl]
                        a0 = v0 * v0
                        a1 = v1 * v1
                        a2 = v2 * v2
                        a3 = v3 * v3
                        for c in range(4, _CH, 4):
                            v0 = buf[c, dl, sl]
                            v1 = buf[c + 1, dl, sl]
                            v2 = buf[c + 2, dl, sl]
                            v3 = buf[c + 3, dl, sl]
                            a0 = a0 + v0 * v0
                            a1 = a1 + v1 * v1
                            a2 = a2 + v2 * v2
                            a3 = a3 + v3 * v3
                        acc = (a0 + a1) + (a2 + a3)
                        if ch == 0:
                            s_buf[d, sl] = acc
                        else:
                            s_buf[d, sl] = s_buf[d, sl] + acc
                return c3

            lax.fori_loop(0, _DS // 2, dl_body, 0)

        def capture(bufA, bufB, j):
            d0 = j * _DS
            for g in range(_NG):
                sl = pl.ds(g * 16, 16)
                rows = lanes + g * 16
                ss = [s_buf[d0 + dl, sl] for dl in range(_DS)]
                sm = jnp.maximum(jnp.maximum(jnp.maximum(ss[0], ss[1]),
                                             jnp.maximum(ss[2], ss[3])),
                                 jnp.maximum(jnp.maximum(ss[4], ss[5]),
                                             jnp.maximum(ss[6], ss[7])))
                sidx = jnp.full((16,), _DS - 1, jnp.int32)
                for dl in range(_DS - 2, -1, -1):
                    sidx = jnp.where(ss[dl] == sm, dl, sidx)
                mo = m_buf[g, :]
                better = sm > mo
                m_buf[g, :] = jnp.where(better, sm, mo)
                go = gi_buf[g, :]
                gi_buf[g, :] = jnp.where(better, sidx + d0, go)
                for c in range(_CH):
                    cfull = jnp.full((16,), c, jnp.int32)
                    v = plsc.load_gather(bufA, [cfull, sidx, rows])
                    bo = best_buf[c, sl]
                    best_buf[c, sl] = jnp.where(better, v, bo)
                for c in range(_CH):
                    cfull = jnp.full((16,), c, jnp.int32)
                    v = plsc.load_gather(bufB, [cfull, sidx, rows])
                    bo = best_buf[_CH + c, sl]
                    best_buf[_CH + c, sl] = jnp.where(better, v, bo)

        def chunk(i, carry):
            b0 = base + i * _NB

            for g in range(_NG):
                m_buf[g, :] = jnp.full((16,), -1.0, jnp.float32)

            pltpu.async_copy(a_src(b0, 0, 0), A0, s0)
            pltpu.async_copy(a_src(b0, 0, 1), A1, s1)
            pltpu.async_copy(a_src(b0, 1, 0), A2, s2)
            pltpu.async_copy(a_src(b0, 1, 1), A3, s3)

            def aj(jj, carry2):
                j0 = 2 * jj
                j1 = j0 + 1
                pltpu.make_async_copy(a_src(b0, j0, 0), A0, s0).wait()
                half(A0, j0, 0)
                pltpu.make_async_copy(a_src(b0, j0, 1), A1, s1).wait()
                half(A1, j0, 1)
                capture(A0, A1, j0)

                @pl.when(jj < nd // 2 - 1)
                def _():
                    pltpu.async_copy(a_src(b0, j0 + 2, 0), A0, s0)
                    pltpu.async_copy(a_src(b0, j0 + 2, 1), A1, s1)

                pltpu.make_async_copy(a_src(b0, j1, 0), A2, s2).wait()
                half(A2, j1, 0)
                pltpu.make_async_copy(a_src(b0, j1, 1), A3, s3).wait()
                half(A3, j1, 1)
                capture(A2, A3, j1)

                @pl.when(jj < nd // 2 - 1)
                def _():
                    pltpu.async_copy(a_src(b0, j1 + 2, 0), A2, s2)
                    pltpu.async_copy(a_src(b0, j1 + 2, 1), A3, s3)

                return carry2

            lax.fori_loop(0, nd // 2, aj, 0)

            def nl(d, c3):
                for g in range(_NG):
                    sl = pl.ds(g * 16, 16)
                    acc = s_buf[d, sl]
                    iv = lax.bitcast_convert_type(acc, jnp.int32)
                    y = lax.bitcast_convert_type(
                        jnp.int32(0x5F3759DF) - (iv >> 1), jnp.float32)
                    for _ in range(2):
                        y = y * (1.5 - 0.5 * acc * y * y)
                    s_buf[d, sl] = jnp.where(acc > 0.0, acc * y, 0.0)
                return c3

            lax.fori_loop(0, D, nl, 0)
            pltpu.sync_copy(s_buf, logt_hbm.at[:, pl.ds(b0, _NB)])

            def fill_nb(nb, k):
                for cl in range(2):
                    for g in range(_NG):
                        sl = pl.ds(g * 16, 16)
                        giv = gi_buf[g, :]
                        rows = lanes + g * 16
                        v = best_buf[2 * k + cl, sl]
                        plsc.store_scatter(nb, [rows, giv + cl * D], v)

            def bj(kk, carry2):
                k0 = 2 * kk

                @pl.when(kk > 0)
                def _():
                    pltpu.make_async_copy(nb0, lat_dst(b0, k0 - 2),
                                          so0).wait()

                fill_nb(nb0, k0)
                pltpu.async_copy(nb0, lat_dst(b0, k0), so0)

                @pl.when(kk > 0)
                def _():
                    pltpu.make_async_copy(nb1, lat_dst(b0, k0 - 1),
                                          so1).wait()

                fill_nb(nb1, k0 + 1)
                pltpu.async_copy(nb1, lat_dst(b0, k0 + 1), so1)
                return carry2

            lax.fori_loop(0, nblk // 2, bj, 0)

            pltpu.make_async_copy(nb0, lat_dst(b0, nblk - 2), so0).wait()
            pltpu.make_async_copy(nb1, lat_dst(b0, nblk - 1), so1).wait()
            for g in range(_NG):
                giv = gi_buf[g, :]
                rows = lanes + g * 16
                for cl in range(2):
                    plsc.store_scatter(nb0, [rows, giv + cl * D], zero16)
                    plsc.store_scatter(nb1, [rows, giv + cl * D], zero16)
            return carry

        lax.fori_loop(0, b_per_w // _NB, chunk, 0)

    logt, lat = run(xt)
    return (jnp.transpose(logt), lat)


# speedup vs baseline: 1.4727x; 1.1923x over previous
import functools
import jax
import jax.numpy as jnp
from jax import lax
from jax.experimental import pallas as pl
from jax.experimental.pallas import tpu as pltpu
from jax.experimental.pallas import tpu_sc as plsc

_NB = 128
_NG = _NB // 16
_CH = 16
_DS = 8
_CS = 2


def kernel(x):
    B, C, D = x.shape
    F = C * D
    xt = jnp.transpose(x, (1, 2, 0))
    info = plsc.get_sparse_core_info()
    NC, NS = info.num_cores, info.num_subcores
    NW = NC * NS
    b_per_w = B // NW
    nd = D // _DS
    nk = C // _CS
    mesh = plsc.VectorSubcoreMesh(core_axis_name="c", subcore_axis_name="s")

    @functools.partial(
        pl.kernel,
        mesh=mesh,
        out_type=[
            jax.ShapeDtypeStruct((D, B), jnp.float32),
            jax.ShapeDtypeStruct((B, F), jnp.float32),
        ],
        scratch_types=[
            pltpu.VMEM((_CH, _DS, _NB), jnp.float32),
            pltpu.VMEM((_CH, _DS, _NB), jnp.float32),
            pltpu.VMEM((_CS, D, _NB), jnp.float32),
            pltpu.VMEM((_CS, D, _NB), jnp.float32),
            pltpu.VMEM((_NB, _CS * D), jnp.float32),
            pltpu.VMEM((_NB, _CS * D), jnp.float32),
            pltpu.VMEM((D, _NB), jnp.float32),
            pltpu.VMEM((_NG, 16), jnp.int32),
            pltpu.SemaphoreType.DMA,
            pltpu.SemaphoreType.DMA,
            pltpu.SemaphoreType.DMA,
            pltpu.SemaphoreType.DMA,
            pltpu.SemaphoreType.DMA,
            pltpu.SemaphoreType.DMA,
        ],
        compiler_params=pltpu.CompilerParams(
            use_tc_tiling_on_sc=True, needs_layout_passes=False),
    )
    def run(xt_hbm, logt_hbm, lat_hbm, xa0, xa1, xb0, xb1, nb0, nb1,
            s_buf, gi_buf, sa0, sa1, sb0, sb1, so0, so1):
        wid = lax.axis_index("s") * NC + lax.axis_index("c")
        base = wid * b_per_w
        lanes = lax.iota(jnp.int32, 16)
        zero16 = jnp.zeros((16,), jnp.float32)

        def a_src(b0, j, ch):
            return xt_hbm.at[pl.ds(ch * _CH, _CH),
                             pl.ds(j * _DS, _DS), pl.ds(b0, _NB)]

        def b_src(b0, k):
            return xt_hbm.at[pl.ds(k * _CS, _CS), :, pl.ds(b0, _NB)]

        def lat_dst(b0, k):
            return lat_hbm.at[pl.ds(b0, _NB), pl.ds(k * (_CS * D), _CS * D)]

        def znb(r, carry):
            for j in range(_CS * D // 16):
                nb0[r, pl.ds(j * 16, 16)] = zero16
                nb1[r, pl.ds(j * 16, 16)] = zero16
            return carry

        lax.fori_loop(0, _NB, znb, 0)

        def fill_nb(nb, xb):
            for cl in range(_CS):
                clv = jnp.full((16,), cl, jnp.int32)
                for g in range(_NG):
                    giv = gi_buf[g, :]
                    rows = lanes + g * 16
                    vals = plsc.load_gather(xb, [clv, giv, rows])
                    plsc.store_scatter(nb, [rows, giv + cl * D], vals)

        def chunk(i, carry):
            b0 = base + i * _NB

            pltpu.async_copy(a_src(b0, 0, 0), xa0, sa0)

            def half(buf, j, ch):
                d0 = j * _DS

                def dl_body(dl2, c3):
                    for du in range(2):
                        dl = dl2 * 2 + du
                        d = d0 + dl
                        for g in range(_NG):
                            sl = pl.ds(g * 16, 16)
                            v0 = buf[0, dl, sl]
                            v1 = buf[1, dl, sl]
                            v2 = buf[2, dl, sl]
                            v3 = buf[3, dl, sl]
                            a0 = v0 * v0
                            a1 = v1 * v1
                            a2 = v2 * v2
                            a3 = v3 * v3
                            for c in range(4, _CH, 4):
                                v0 = buf[c, dl, sl]
                                v1 = buf[c + 1, dl, sl]
                                v2 = buf[c + 2, dl, sl]
                                v3 = buf[c + 3, dl, sl]
                                a0 = a0 + v0 * v0
                                a1 = a1 + v1 * v1
                                a2 = a2 + v2 * v2
                                a3 = a3 + v3 * v3
                            acc = (a0 + a1) + (a2 + a3)
                            if ch == 0:
                                s_buf[d, sl] = acc
                            else:
                                s_buf[d, sl] = s_buf[d, sl] + acc
                    return c3

                lax.fori_loop(0, _DS // 2, dl_body, 0)

            def aj(j, carry2):
                pltpu.async_copy(a_src(b0, j, 1), xa1, sa1)
                pltpu.make_async_copy(a_src(b0, j, 0), xa0, sa0).wait()
                half(xa0, j, 0)

                @pl.when(j < nd - 1)
                def _():
                    pltpu.async_copy(a_src(b0, j + 1, 0), xa0, sa0)

                pltpu.make_async_copy(a_src(b0, j, 1), xa1, sa1).wait()
                half(xa1, j, 1)
                return carry2

            lax.fori_loop(0, nd, aj, 0)

            pltpu.async_copy(b_src(b0, 0), xb0, sb0)

            for g in range(_NG):
                sl = pl.ds(g * 16, 16)
                ss = [s_buf[d, sl] for d in range(D)]
                m = ss[0]
                for d in range(1, D):
                    m = jnp.maximum(m, ss[d])
                cand = jnp.full((16,), D, jnp.int32)
                for d in range(D - 1, -1, -1):
                    cand = jnp.where(ss[d] == m, d, cand)
                gi_buf[g, :] = cand

            def nl(d, c3):
                for g in range(_NG):
                    sl = pl.ds(g * 16, 16)
                    acc = s_buf[d, sl]
                    iv = lax.bitcast_convert_type(acc, jnp.int32)
                    y = lax.bitcast_convert_type(
                        jnp.int32(0x5F3759DF) - (iv >> 1), jnp.float32)
                    for _ in range(2):
                        y = y * (1.5 - 0.5 * acc * y * y)
                    s_buf[d, sl] = jnp.where(acc > 0.0, acc * y, 0.0)
                return c3

            lax.fori_loop(0, D, nl, 0)
            pltpu.sync_copy(s_buf, logt_hbm.at[:, pl.ds(b0, _NB)])

            def bj(jj, carry2):
                k0 = 2 * jj
                pltpu.async_copy(b_src(b0, k0 + 1), xb1, sb1)
                pltpu.make_async_copy(b_src(b0, k0), xb0, sb0).wait()

                @pl.when(jj > 0)
                def _():
                    pltpu.make_async_copy(nb0, lat_dst(b0, k0 - 2),
                                          so0).wait()

                fill_nb(nb0, xb0)
                pltpu.async_copy(nb0, lat_dst(b0, k0), so0)

                @pl.when(jj < nk // 2 - 1)
                def _():
                    pltpu.async_copy(b_src(b0, k0 + 2), xb0, sb0)

                pltpu.make_async_copy(b_src(b0, k0 + 1), xb1, sb1).wait()

                @pl.when(jj > 0)
                def _():
                    pltpu.make_async_copy(nb1, lat_dst(b0, k0 - 1),
                                          so1).wait()

                fill_nb(nb1, xb1)
                pltpu.async_copy(nb1, lat_dst(b0, k0 + 1), so1)
                return carry2

            lax.fori_loop(0, nk // 2, bj, 0)

            pltpu.make_async_copy(nb0, lat_dst(b0, nk - 2), so0).wait()
            pltpu.make_async_copy(nb1, lat_dst(b0, nk - 1), so1).wait()
            for g in range(_NG):
                giv = gi_buf[g, :]
                rows = lanes + g * 16
                for cl in range(_CS):
                    plsc.store_scatter(nb0, [rows, giv + cl * D], zero16)
                    plsc.store_scatter(nb1, [rows, giv + cl * D], zero16)
            return carry

        lax.fori_loop(0, b_per_w // _NB, chunk, 0)

    logt, lat = run(xt)
    return (jnp.transpose(logt), lat)
